# baseline (speedup 1.0000x reference)
import functools

import jax
import jax.numpy as jnp
from jax import lax
from jax.experimental import pallas as pl
from jax.experimental.pallas import tpu as pltpu, tpu_sc as plsc

ND = 38300
NM = 49500
N = ND + NM
H = 4
F = 64
FH = F * H
W = 272
RN = 3600
NRANGES = 26
NPAD = NRANGES * RN
RPC = NRANGES // 2
SLOPE = 0.2
BR = 200
NBLK = N // BR
KB = 64
SC_CH = 4096
CAP = 6400
CAPA = CAP + 256
ZB = 12
NPAIR = 16384
EPS = 1e-16


def _iota16():
    return lax.iota(jnp.int32, 16)


def _prep_body(d_ref, m_ref, wda_ref, wma_ref, wdb_ref, wmb_ref,
               a1a_ref, a2a_ref, a1b_ref, a2b_ref,
               hxa_ref, s2a_ref, hxb_ref, s2b_ref):
    i = pl.program_id(0)
    row = i * BR + lax.broadcasted_iota(jnp.int32, (BR, 1), 0)
    isd = row < ND
    d = d_ref[...]
    m = m_ref[...]
    zpad = jnp.zeros((BR, W - FH - H), jnp.float32)
    for wd, wm, a1, a2, hx_ref, s2_ref in (
        (wda_ref, wma_ref, a1a_ref, a2a_ref, hxa_ref, s2a_ref),
        (wdb_ref, wmb_ref, a1b_ref, a2b_ref, hxb_ref, s2b_ref),
    ):
        hd = jnp.dot(d, wd[...], preferred_element_type=jnp.float32)
        hm = jnp.dot(m, wm[...], preferred_element_type=jnp.float32)
        h = jnp.where(isd, hd, hm)
        s1 = jnp.dot(h, a1[...], preferred_element_type=jnp.float32)
        s2 = jnp.dot(h, a2[...], preferred_element_type=jnp.float32)
        hx_ref[...] = jnp.concatenate([h, s1, zpad], axis=1)
        s2_ref[...] = s2


def _prep(d_sim, m_sim, wda, wma, wdb, wmb, a1a, a2a, a1b, a2b):
    full = lambda shp: pl.BlockSpec(shp, lambda i: (0, 0))
    return pl.pallas_call(
        _prep_body,
        grid=(NBLK,),
        in_specs=[
            pl.BlockSpec((BR, 128), lambda i: (i, 0)),
            pl.BlockSpec((BR, 128), lambda i: (i, 0)),
            full((128, FH)), full((128, FH)), full((128, FH)), full((128, FH)),
            full((FH, H)), full((FH, H)), full((FH, H)), full((FH, H)),
        ],
        out_specs=[
            pl.BlockSpec((BR, W), lambda i: (i, 0)),
            pl.BlockSpec((BR, H), lambda i: (i, 0)),
            pl.BlockSpec((BR, W), lambda i: (i, 0)),
            pl.BlockSpec((BR, H), lambda i: (i, 0)),
        ],
        out_shape=[
            jax.ShapeDtypeStruct((N, W), jnp.float32),
            jax.ShapeDtypeStruct((N, H), jnp.float32),
            jax.ShapeDtypeStruct((N, W), jnp.float32),
            jax.ShapeDtypeStruct((N, H), jnp.float32),
        ],
    )(d_sim, m_sim, wda, wma, wdb, wmb, a1a, a2a, a1b, a2b)


def _gat_body_make(E, NG):
    NCH = pl.cdiv(E, SC_CH)
    NJ = pl.cdiv(NCH, 16)
    last_off = E - SC_CH

    def body(hx_ref, s2_ref, src_ref, dst_ref, zz_ref, u_ref,
             sbuf, dbuf, pkc, s2blk, hb0, hb1, idxb0, idxb1, ofsb0, ofsb1,
             sem0, sem1, sem2, sem3, u_acc):
        cid = lax.axis_index("c")
        sid = lax.axis_index("s")
        neg1 = jnp.full((16,), -1, jnp.int32)

        def range_body(t, _):
            gi = t // RPC
            ri = t - gi * RPC
            ebase = gi * E
            ubase = gi * NPAD
            rg = cid * RPC + ri
            base = rg * RN
            row0 = sid * (RN // 16)
            pltpu.sync_copy(zz_ref.at[pl.ds(0, RN // 16)],
                            u_acc.at[pl.ds(row0, RN // 16)])
            pltpu.sync_copy(s2_ref.at[pl.ds(base * H, RN * H)], s2blk)

            def chunk_body(j, k):
                chunk = j * 16 + sid
                cs = chunk * SC_CH
                off = jnp.minimum(cs, last_off)
                pltpu.sync_copy(src_ref.at[pl.ds(ebase + off, SC_CH)], sbuf)
                pltpu.sync_copy(dst_ref.at[pl.ds(ebase + off, SC_CH)], dbuf)
                lo16 = jnp.minimum(cs - off, SC_CH) // 16

                def pre(t, _):
                    dbuf[pl.ds(t * 16, 16)] = neg1
                    return 0

                lax.fori_loop(0, lo16, pre, 0)

                def group_body(g, k):
                    srcv = sbuf[pl.ds(g * 16, 16)]
                    dstv = dbuf[pl.ds(g * 16, 16)]
                    msk = (dstv >= base) & (dstv < base + RN)
                    inc = jnp.where(msk, jnp.int32(1), jnp.int32(0))
                    kc = jnp.minimum(k, CAP)
                    packed = srcv * 8192 + (dstv - base)
                    _, vs = plsc.sort_key_val(1 - inc, packed)
                    pkc[pl.ds(kc, 16)] = vs
                    return k + jnp.sum(inc)

                return lax.fori_loop(0, SC_CH // 16, group_body, k)

            k = lax.fori_loop(0, NJ, chunk_body, jnp.int32(0))
            k = jnp.minimum(k, CAP)
            plsc.subcore_barrier()

            bufs = ((hb0, idxb0, ofsb0, sem0, sem2),
                    (hb1, idxb1, ofsb1, sem1, sem3))

            def build(b0, idxb, ofsb):
                for g in range(KB // 16):
                    pos = b0 + g * 16 + _iota16()
                    valid = pos < k
                    pv = pkc[pl.ds(b0 + g * 16, 16)]
                    sv = lax.shift_right_logical(pv, 13)
                    ov = jnp.bitwise_and(pv, 8191)
                    idxb[pl.ds(g * 16, 16)] = jnp.where(valid, sv, 0)
                    ofsb[pl.ds(g * 16, 16)] = jnp.where(valid, ov, 0)

            def process(b0, hb, ofsb, ssem):
                for g in range(KB // 16):
                    rowv = g * 16 + _iota16()
                    valid = (b0 + rowv) < k
                    ov = ofsb[pl.ds(g * 16, 16)]
                    for hd in range(H):
                        colv = jnp.full((16,), FH + hd, jnp.int32)
                        s1v = plsc.load_gather(hb, [rowv, colv])
                        s2v = plsc.load_gather(s2blk, [ov * H + hd])
                        e = s1v + s2v
                        e = jnp.where(e >= 0, e, SLOPE * e)
                        eev = jnp.where(valid, jnp.exp(e), 0.0)
                        plsc.store_scatter(hb, [rowv, colv], eev)

                def scale_body(r, _):
                    ev = hb[r, pl.ds(FH, 16)]
                    for hd in range(H):
                        ee = ev[hd]
                        for c in range(hd * 4, hd * 4 + 4):
                            hb[r, pl.ds(c * 16, 16)] = (
                                hb[r, pl.ds(c * 16, 16)] * ee)
                    return 0

                lax.fori_loop(0, KB, scale_body, 0)
                return pltpu.async_copy(hb, u_acc.at[ofsb], ssem, add=True)

            def pair_body(i, _):
                descs = []
                sdescs = []
                nb = len(bufs)
                for par, (hb, idxb, ofsb, sem, ssem) in enumerate(bufs):
                    b0 = (nb * i + par) * KB
                    build(b0, idxb, ofsb)
                    descs.append(pltpu.async_copy(hx_ref.at[idxb], hb, sem))
                for par, (hb, idxb, ofsb, sem, ssem) in enumerate(bufs):
                    b0 = (nb * i + par) * KB
                    descs[par].wait()
                    sdescs.append(process(b0, hb, ofsb, ssem))
                for sd in sdescs:
                    sd.wait()
                return 0

            lax.fori_loop(0, pl.cdiv(k, len(bufs) * KB), pair_body, 0)
            plsc.subcore_barrier()
            for o in range(0, RN // 16, 2 * KB):
                sz = min(2 * KB, RN // 16 - o)
                pltpu.sync_copy(u_acc.at[pl.ds(row0 + o, sz)],
                                u_ref.at[pl.ds(ubase + base + row0 + o, sz)])
            return 0

        lax.fori_loop(0, NG * RPC, range_body, 0)

    return body


def _gat_edges(hx, s2flat, src, dst, zz, ng=1):
    E = src.shape[0] // ng
    mesh = plsc.VectorSubcoreMesh(core_axis_name="c", subcore_axis_name="s")
    kern = pl.kernel(
        _gat_body_make(E, ng),
        out_type=jax.ShapeDtypeStruct((ng * NPAD, W), jnp.float32),
        mesh=mesh,
        compiler_params=pltpu.CompilerParams(
            needs_layout_passes=False, use_tc_tiling_on_sc=False),
        scratch_types=[
            pltpu.VMEM((SC_CH,), jnp.int32),
            pltpu.VMEM((SC_CH,), jnp.int32),
            pltpu.VMEM((CAPA,), jnp.int32),
            pltpu.VMEM((RN * H,), jnp.float32),
            pltpu.VMEM((KB, W), jnp.float32),
            pltpu.VMEM((KB, W), jnp.float32),
            pltpu.VMEM((KB,), jnp.int32),
            pltpu.VMEM((KB,), jnp.int32),
            pltpu.VMEM((KB,), jnp.int32),
            pltpu.VMEM((KB,), jnp.int32),
            pltpu.SemaphoreType.DMA,
            pltpu.SemaphoreType.DMA,
            pltpu.SemaphoreType.DMA,
            pltpu.SemaphoreType.DMA,
            pltpu.VMEM_SHARED((RN, W), jnp.float32),
        ],
    )
    return kern(hx, s2flat, src, dst, zz)


def _posta_body(u0, u1, u2, u3, u4, ws1_ref, bs1_ref, ws2_ref, ts_ref):
    i = pl.program_id(0)

    @pl.when(i == 0)
    def _():
        ts_ref[...] = jnp.zeros((8, 128), jnp.float32)

    row = i * BR + lax.broadcasted_iota(jnp.int32, (BR, 1), 0)
    isd = row < ND
    ws1 = ws1_ref[...]
    bs1 = bs1_ref[...]
    ws2 = ws2_ref[...]
    lane = lax.broadcasted_iota(jnp.int32, (8, 128), 1)
    prow = lax.broadcasted_iota(jnp.int32, (8, 128), 0)
    acc = jnp.zeros((8, 128), jnp.float32)
    for p, u_ref in enumerate((u0, u1, u2, u3, u4)):
        y = _u_to_y(u_ref[...])
        t = jnp.dot(jnp.tanh(jnp.dot(y, ws1,
                                     preferred_element_type=jnp.float32)
                             + bs1),
                    ws2, preferred_element_type=jnp.float32)
        td = jnp.sum(jnp.where(isd, t, 0.0))
        tm = jnp.sum(jnp.where(isd, 0.0, t))
        val = jnp.where(lane == 0, td, jnp.where(lane == 1, tm, 0.0))
        acc = acc + jnp.where(prow == p, val, 0.0)
    ts_ref[...] = ts_ref[...] + acc


def _u_to_y(u):
    s = u[:, FH:FH + H]
    den = jnp.concatenate(
        [jnp.broadcast_to(s[:, hd:hd + 1], (BR, F)) for hd in range(H)],
        axis=1) + EPS
    y = u[:, :FH] / den
    return jnp.where(y > 0, y, jnp.exp(y) - 1.0)


def _u_specs():
    sh = NPAD // BR
    specs = [pl.BlockSpec((BR, W), lambda i: (i, 0))]
    for gi in range(4):
        specs.append(pl.BlockSpec((BR, W), lambda i, g=gi: (g * sh + i, 0)))
    return specs


def _posta(u0, u4, ws1, bs1, ws2):
    full2 = lambda shp: pl.BlockSpec(shp, lambda i: (0, 0))
    return pl.pallas_call(
        _posta_body,
        grid=(NBLK,),
        in_specs=_u_specs() + [full2((FH, 128)),
                               pl.BlockSpec((1, 128), lambda i: (0, 0)),
                               full2((128, 1))],
        out_specs=full2((8, 128)),
        out_shape=jax.ShapeDtypeStruct((8, 128), jnp.float32),
    )(u0, u4, u4, u4, u4, ws1, bs1.reshape(1, 128), ws2)


def _postb_body(u0, u1, u2, u3, u4, d_ref, m_ref, ts_ref,
                wdfc_ref, bdfc_ref, wmfc_ref, bmfc_ref, whfc_ref, bhfc_ref,
                h_ref):
    i = pl.program_id(0)
    row = i * BR + lax.broadcasted_iota(jnp.int32, (BR, 1), 0)
    isd = row < ND
    ts = ts_ref[...]
    h1 = jnp.zeros((BR, FH), jnp.float32)
    for p, u_ref in enumerate((u0, u1, u2, u3, u4)):
        betad = jax.nn.sigmoid(ts[p, 0] / ND)
        betam = jax.nn.sigmoid(ts[p, 1] / NM)
        beta = jnp.where(isd, betad, betam)
        h1 = h1 + beta * _u_to_y(u_ref[...])
    sim = jnp.where(isd, d_ref[...], m_ref[...])
    fd = (jnp.dot(h1, wdfc_ref[:FH, :], preferred_element_type=jnp.float32)
          + jnp.dot(sim, wdfc_ref[FH:, :], preferred_element_type=jnp.float32)
          + bdfc_ref[...])
    fm = (jnp.dot(h1, wmfc_ref[:FH, :], preferred_element_type=jnp.float32)
          + jnp.dot(sim, wmfc_ref[FH:, :], preferred_element_type=jnp.float32)
          + bmfc_ref[...])
    f = jnp.where(isd, fd, fm)
    f = jnp.where(f > 0, f, jnp.exp(f) - 1.0)
    h = (jnp.dot(f, whfc_ref[...], preferred_element_type=jnp.float32)
         + bhfc_ref[...])
    h_ref[...] = jnp.where(h > 0, h, jnp.exp(h) - 1.0)


def _postb(u0, u4, d_sim, m_sim, ts, wdfc, bdfc, wmfc, bmfc, whfc, bhfc):
    full2 = lambda shp: pl.BlockSpec(shp, lambda i: (0, 0))
    return pl.pallas_call(
        _postb_body,
        grid=(NBLK,),
        in_specs=_u_specs()
        + [pl.BlockSpec((BR, 128), lambda i: (i, 0)),
           pl.BlockSpec((BR, 128), lambda i: (i, 0)),
           full2((8, 128)),
           full2((FH + 128, F)), pl.BlockSpec((1, F), lambda i: (0, 0)),
           full2((FH + 128, F)), pl.BlockSpec((1, F), lambda i: (0, 0)),
           full2((F, F)), pl.BlockSpec((1, F), lambda i: (0, 0))],
        out_specs=pl.BlockSpec((BR, F), lambda i: (i, 0)),
        out_shape=jax.ShapeDtypeStruct((N, F), jnp.float32),
    )(u0, u4, u4, u4, u4, d_sim, m_sim, ts, wdfc, bdfc.reshape(1, F), wmfc,
      bmfc.reshape(1, F), whfc, bhfc.reshape(1, F))


def _final_body(h_ref, dis_ref, mir_ref, wp_ref, out_ref,
                didx, midx, hd, hm, ob, wbuf):
    cid = lax.axis_index("c")
    sid = lax.axis_index("s")
    wid = sid * 2 + cid
    per = NPAIR // 32
    base = wid * per
    pltpu.sync_copy(dis_ref.at[pl.ds(base, per)], didx)
    pltpu.sync_copy(mir_ref.at[pl.ds(base, per)], midx)
    pltpu.sync_copy(wp_ref, wbuf)
    pltpu.sync_copy(h_ref.at[didx], hd)
    pltpu.sync_copy(h_ref.at[midx], hm)
    wv = [wbuf[pl.ds(c * 16, 16)] for c in range(8)]
    bp = wbuf[pl.ds(128, 16)][0]

    def grp(g, _):
        gv = jnp.zeros((16,), jnp.float32)
        for kk in range(16):
            p = g * 16 + kk
            acc = hd[p, pl.ds(0, 16)] * wv[0]
            for c in range(1, 4):
                acc = acc + hd[p, pl.ds(c * 16, 16)] * wv[c]
            for c in range(4):
                acc = acc + hm[p, pl.ds(c * 16, 16)] * wv[4 + c]
            sc = jnp.sum(acc)
            gv = jnp.where(_iota16() == kk, sc, gv)
        ev = jnp.exp(-(gv + bp))
        ob[pl.ds(g * 16, 16)] = 1.0 / (1.0 + ev)
        return 0

    lax.fori_loop(0, per // 16, grp, 0)
    pltpu.sync_copy(ob, out_ref.at[pl.ds(base, per)])


def _final(h, diseases, mirnas, wpb):
    per = NPAIR // 32
    mesh = plsc.VectorSubcoreMesh(core_axis_name="c", subcore_axis_name="s")
    kern = pl.kernel(
        _final_body,
        out_type=jax.ShapeDtypeStruct((NPAIR,), jnp.float32),
        mesh=mesh,
        compiler_params=pltpu.CompilerParams(
            needs_layout_passes=False, use_tc_tiling_on_sc=False),
        scratch_types=[
            pltpu.VMEM((per,), jnp.int32),
            pltpu.VMEM((per,), jnp.int32),
            pltpu.VMEM((per, F), jnp.float32),
            pltpu.VMEM((per, F), jnp.float32),
            pltpu.VMEM((per,), jnp.float32),
            pltpu.VMEM((144,), jnp.float32),
        ],
    )
    return kern(h, diseases, mirnas, wpb)


def kernel(d_sim, m_sim, Wd, Wm, a1, a2, Wmd, Wmm, am1, am2, Ws1, bs1, Ws2,
           Wmfc, bmfc, Wdfc, bdfc, Whfc, bhfc, Wp, bp,
           edge_index0, edge_index_c, edge_index_e, edge_index_t,
           edge_index_g, diseases, mirnas):
    f32 = jnp.float32
    cat = lambda w: w.astype(f32).transpose(1, 0, 2).reshape(128, FH)
    eye = jnp.eye(H, dtype=f32)
    blk = lambda a: (a.astype(f32)[:, :, None] * eye[:, None, :]).reshape(FH, H)
    wda, wma = cat(Wd), cat(Wm)
    wdb, wmb = cat(Wmd), cat(Wmm)
    a1a, a2a = blk(a1), blk(a2)
    a1b, a2b = blk(am1), blk(am2)

    hxa, s2a, hxb, s2b = _prep(d_sim, m_sim, wda, wma, wdb, wmb,
                               a1a, a2a, a1b, a2b)
    pad = lambda s: jnp.pad(s, ((0, NPAD - N), (0, 0))).reshape(-1)
    s2a_f, s2b_f = pad(s2a), pad(s2b)

    zz = jnp.zeros((RN // 16, W), jnp.float32)
    e0 = edge_index0.astype(jnp.int32)
    u0 = _gat_edges(hxa, s2a_f, e0[0], e0[1], zz)
    ecat = jnp.concatenate([edge_index_c, edge_index_e, edge_index_t,
                            edge_index_g], axis=1).astype(jnp.int32)
    u4 = _gat_edges(hxb, s2b_f, ecat[0], ecat[1], zz, ng=4)

    ts = _posta(u0, u4, Ws1, bs1, Ws2)
    h = _postb(u0, u4, d_sim, m_sim, ts, Wdfc, bdfc, Wmfc, bmfc, Whfc, bhfc)

    wpb = jnp.concatenate([Wp.reshape(-1), bp.reshape(-1),
                           jnp.zeros((15,), f32)])
    out = _final(h, diseases.astype(jnp.int32), mirnas.astype(jnp.int32), wpb)
    return out.reshape(NPAIR, 1)


# speedup vs baseline: 1.0387x; 1.0387x over previous
import functools

import jax
import jax.numpy as jnp
from jax import lax
from jax.experimental import pallas as pl
from jax.experimental.pallas import tpu as pltpu, tpu_sc as plsc

ND = 38300
NM = 49500
N = ND + NM
H = 4
F = 64
FH = F * H
W = 272
RN = 3680
NRANGES = 24
NPAD = NRANGES * RN
RPC = NRANGES // 2
SLOPE = 0.2
BR = 200
NBLK = N // BR
KB = 64
SC_CH = 6144
CAP = 6144
CAPA = CAP + 128
ZB = 12
NPAIR = 16384
EPS = 1e-16


def _iota16():
    return lax.iota(jnp.int32, 16)


def _prep_body(d_ref, m_ref, wda_ref, wma_ref, wdb_ref, wmb_ref,
               a1a_ref, a2a_ref, a1b_ref, a2b_ref,
               hxa_ref, s2a_ref, hxb_ref, s2b_ref):
    i = pl.program_id(0)
    row = i * BR + lax.broadcasted_iota(jnp.int32, (BR, 1), 0)
    isd = row < ND
    d = d_ref[...]
    m = m_ref[...]
    zpad = jnp.zeros((BR, W - FH - H), jnp.float32)
    for wd, wm, a1, a2, hx_ref, s2_ref in (
        (wda_ref, wma_ref, a1a_ref, a2a_ref, hxa_ref, s2a_ref),
        (wdb_ref, wmb_ref, a1b_ref, a2b_ref, hxb_ref, s2b_ref),
    ):
        hd = jnp.dot(d, wd[...], preferred_element_type=jnp.float32)
        hm = jnp.dot(m, wm[...], preferred_element_type=jnp.float32)
        h = jnp.where(isd, hd, hm)
        s1 = jnp.dot(h, a1[...], preferred_element_type=jnp.float32)
        s2 = jnp.dot(h, a2[...], preferred_element_type=jnp.float32)
        hx_ref[...] = jnp.concatenate([h, s1, zpad], axis=1)
        s2_ref[...] = s2


def _prep(d_sim, m_sim, wda, wma, wdb, wmb, a1a, a2a, a1b, a2b):
    full = lambda shp: pl.BlockSpec(shp, lambda i: (0, 0))
    return pl.pallas_call(
        _prep_body,
        grid=(NBLK,),
        in_specs=[
            pl.BlockSpec((BR, 128), lambda i: (i, 0)),
            pl.BlockSpec((BR, 128), lambda i: (i, 0)),
            full((128, FH)), full((128, FH)), full((128, FH)), full((128, FH)),
            full((FH, H)), full((FH, H)), full((FH, H)), full((FH, H)),
        ],
        out_specs=[
            pl.BlockSpec((BR, W), lambda i: (i, 0)),
            pl.BlockSpec((BR, H), lambda i: (i, 0)),
            pl.BlockSpec((BR, W), lambda i: (i, 0)),
            pl.BlockSpec((BR, H), lambda i: (i, 0)),
        ],
        out_shape=[
            jax.ShapeDtypeStruct((N, W), jnp.float32),
            jax.ShapeDtypeStruct((N, H), jnp.float32),
            jax.ShapeDtypeStruct((N, W), jnp.float32),
            jax.ShapeDtypeStruct((N, H), jnp.float32),
        ],
    )(d_sim, m_sim, wda, wma, wdb, wmb, a1a, a2a, a1b, a2b)


def _gat_body_make(E):
    NCH = pl.cdiv(E, SC_CH)
    NJ = pl.cdiv(NCH, 16)
    last_off = E - SC_CH

    def body(hx_ref, s2_ref, src_ref, dst_ref, zz_ref, u_ref,
             sbuf, dbuf, pkc, s2blk, hb0, hb1, idxb0, idxb1, ofsb0, ofsb1,
             sem0, sem1, sem2, sem3, u_acc):
        cid = lax.axis_index("c")
        sid = lax.axis_index("s")
        neg1 = jnp.full((16,), -1, jnp.int32)

        def range_body(ri, _):
            rg = cid * RPC + ri
            base = rg * RN
            row0 = sid * (RN // 16)
            pltpu.sync_copy(zz_ref.at[pl.ds(0, RN // 16)],
                            u_acc.at[pl.ds(row0, RN // 16)])
            pltpu.sync_copy(s2_ref.at[pl.ds(base * H, RN * H)], s2blk)

            def chunk_body(j, k):
                chunk = j * 16 + sid
                cs = chunk * SC_CH
                off = jnp.minimum(cs, last_off)
                pltpu.sync_copy(src_ref.at[pl.ds(off, SC_CH)], sbuf)
                pltpu.sync_copy(dst_ref.at[pl.ds(off, SC_CH)], dbuf)
                lo16 = jnp.minimum(cs - off, SC_CH) // 16

                def pre(t, _):
                    dbuf[pl.ds(t * 16, 16)] = neg1
                    return 0

                lax.fori_loop(0, lo16, pre, 0)

                def group_body(g, k):
                    srcv = sbuf[pl.ds(g * 16, 16)]
                    dstv = dbuf[pl.ds(g * 16, 16)]
                    msk = (dstv >= base) & (dstv < base + RN)
                    inc = jnp.where(msk, jnp.int32(1), jnp.int32(0))
                    kc = jnp.minimum(k, CAP)
                    packed = srcv * 8192 + (dstv - base)
                    _, vs = plsc.sort_key_val(1 - inc, packed)
                    pkc[pl.ds(kc, 16)] = vs
                    return k + jnp.sum(inc)

                return lax.fori_loop(0, SC_CH // 16, group_body, k)

            k = lax.fori_loop(0, NJ, chunk_body, jnp.int32(0))
            k = jnp.minimum(k, CAP)
            plsc.subcore_barrier()

            bufs = ((hb0, idxb0, ofsb0, sem0, sem2),
                    (hb1, idxb1, ofsb1, sem1, sem3))

            def build(b0, idxb, ofsb):
                for g in range(KB // 16):
                    pos = b0 + g * 16 + _iota16()
                    valid = pos < k
                    pv = pkc[pl.ds(b0 + g * 16, 16)]
                    sv = lax.shift_right_logical(pv, 13)
                    ov = jnp.bitwise_and(pv, 8191)
                    idxb[pl.ds(g * 16, 16)] = jnp.where(valid, sv, 0)
                    ofsb[pl.ds(g * 16, 16)] = jnp.where(valid, ov, 0)

            def process(b0, hb, ofsb, ssem):
                for g in range(KB // 16):
                    rowv = g * 16 + _iota16()
                    valid = (b0 + rowv) < k
                    ov = ofsb[pl.ds(g * 16, 16)]
                    for hd in range(H):
                        colv = jnp.full((16,), FH + hd, jnp.int32)
                        s1v = plsc.load_gather(hb, [rowv, colv])
                        s2v = plsc.load_gather(s2blk, [ov * H + hd])
                        e = s1v + s2v
                        e = jnp.where(e >= 0, e, SLOPE * e)
                        eev = jnp.where(valid, jnp.exp(e), 0.0)
                        plsc.store_scatter(hb, [rowv, colv], eev)

                def scale_body(r, _):
                    ev = hb[r, pl.ds(FH, 16)]
                    for hd in range(H):
                        ee = ev[hd]
                        for c in range(hd * 4, hd * 4 + 4):
                            hb[r, pl.ds(c * 16, 16)] = (
                                hb[r, pl.ds(c * 16, 16)] * ee)
                    return 0

                lax.fori_loop(0, KB, scale_body, 0)
                return pltpu.async_copy(hb, u_acc.at[ofsb], ssem, add=True)

            def pair_body(i, _):
                descs = []
                sdescs = []
                nb = len(bufs)
                for par, (hb, idxb, ofsb, sem, ssem) in enumerate(bufs):
                    b0 = (nb * i + par) * KB
                    build(b0, idxb, ofsb)
                    descs.append(pltpu.async_copy(hx_ref.at[idxb], hb, sem))
                for par, (hb, idxb, ofsb, sem, ssem) in enumerate(bufs):
                    b0 = (nb * i + par) * KB
                    descs[par].wait()
                    sdescs.append(process(b0, hb, ofsb, ssem))
                for sd in sdescs:
                    sd.wait()
                return 0

            lax.fori_loop(0, pl.cdiv(k, len(bufs) * KB), pair_body, 0)
            plsc.subcore_barrier()
            for o in range(0, RN // 16, 2 * KB):
                sz = min(2 * KB, RN // 16 - o)
                pltpu.sync_copy(u_acc.at[pl.ds(row0 + o, sz)],
                                u_ref.at[pl.ds(base + row0 + o, sz)])
            return 0

        lax.fori_loop(0, RPC, range_body, 0)

    return body


def _gat_edges(hx, s2flat, src, dst, zz):
    E = src.shape[0]
    mesh = plsc.VectorSubcoreMesh(core_axis_name="c", subcore_axis_name="s")
    kern = pl.kernel(
        _gat_body_make(E),
        out_type=jax.ShapeDtypeStruct((NPAD, W), jnp.float32),
        mesh=mesh,
        compiler_params=pltpu.CompilerParams(
            needs_layout_passes=False, use_tc_tiling_on_sc=False),
        scratch_types=[
            pltpu.VMEM((SC_CH,), jnp.int32),
            pltpu.VMEM((SC_CH,), jnp.int32),
            pltpu.VMEM((CAPA,), jnp.int32),
            pltpu.VMEM((RN * H,), jnp.float32),
            pltpu.VMEM((KB, W), jnp.float32),
            pltpu.VMEM((KB, W), jnp.float32),
            pltpu.VMEM((KB,), jnp.int32),
            pltpu.VMEM((KB,), jnp.int32),
            pltpu.VMEM((KB,), jnp.int32),
            pltpu.VMEM((KB,), jnp.int32),
            pltpu.SemaphoreType.DMA,
            pltpu.SemaphoreType.DMA,
            pltpu.SemaphoreType.DMA,
            pltpu.SemaphoreType.DMA,
            pltpu.VMEM_SHARED((RN, W), jnp.float32),
        ],
    )
    return kern(hx, s2flat, src, dst, zz)


def _posta_body(u0, u1, u2, u3, u4, ws1_ref, bs1_ref, ws2_ref, ts_ref):
    i = pl.program_id(0)

    @pl.when(i == 0)
    def _():
        ts_ref[...] = jnp.zeros((8, 128), jnp.float32)

    row = i * BR + lax.broadcasted_iota(jnp.int32, (BR, 1), 0)
    isd = row < ND
    ws1 = ws1_ref[...]
    bs1 = bs1_ref[...]
    ws2 = ws2_ref[...]
    lane = lax.broadcasted_iota(jnp.int32, (8, 128), 1)
    prow = lax.broadcasted_iota(jnp.int32, (8, 128), 0)
    acc = jnp.zeros((8, 128), jnp.float32)
    for p, u_ref in enumerate((u0, u1, u2, u3, u4)):
        y = _u_to_y(u_ref[...])
        t = jnp.dot(jnp.tanh(jnp.dot(y, ws1,
                                     preferred_element_type=jnp.float32)
                             + bs1),
                    ws2, preferred_element_type=jnp.float32)
        td = jnp.sum(jnp.where(isd, t, 0.0))
        tm = jnp.sum(jnp.where(isd, 0.0, t))
        val = jnp.where(lane == 0, td, jnp.where(lane == 1, tm, 0.0))
        acc = acc + jnp.where(prow == p, val, 0.0)
    ts_ref[...] = ts_ref[...] + acc


def _u_to_y(u):
    s = u[:, FH:FH + H]
    den = jnp.concatenate(
        [jnp.broadcast_to(s[:, hd:hd + 1], (BR, F)) for hd in range(H)],
        axis=1) + EPS
    y = u[:, :FH] / den
    return jnp.where(y > 0, y, jnp.exp(y) - 1.0)


def _posta(us, ws1, bs1, ws2):
    full2 = lambda shp: pl.BlockSpec(shp, lambda i: (0, 0))
    ublk = pl.BlockSpec((BR, W), lambda i: (i, 0))
    return pl.pallas_call(
        _posta_body,
        grid=(NBLK,),
        in_specs=[ublk] * 5 + [full2((FH, 128)),
                               pl.BlockSpec((1, 128), lambda i: (0, 0)),
                               full2((128, 1))],
        out_specs=full2((8, 128)),
        out_shape=jax.ShapeDtypeStruct((8, 128), jnp.float32),
    )(*us, ws1, bs1.reshape(1, 128), ws2)


def _postb_body(u0, u1, u2, u3, u4, d_ref, m_ref, ts_ref,
                wdfc_ref, bdfc_ref, wmfc_ref, bmfc_ref, whfc_ref, bhfc_ref,
                h_ref):
    i = pl.program_id(0)
    row = i * BR + lax.broadcasted_iota(jnp.int32, (BR, 1), 0)
    isd = row < ND
    ts = ts_ref[...]
    h1 = jnp.zeros((BR, FH), jnp.float32)
    for p, u_ref in enumerate((u0, u1, u2, u3, u4)):
        betad = jax.nn.sigmoid(ts[p, 0] / ND)
        betam = jax.nn.sigmoid(ts[p, 1] / NM)
        beta = jnp.where(isd, betad, betam)
        h1 = h1 + beta * _u_to_y(u_ref[...])
    sim = jnp.where(isd, d_ref[...], m_ref[...])
    fd = (jnp.dot(h1, wdfc_ref[:FH, :], preferred_element_type=jnp.float32)
          + jnp.dot(sim, wdfc_ref[FH:, :], preferred_element_type=jnp.float32)
          + bdfc_ref[...])
    fm = (jnp.dot(h1, wmfc_ref[:FH, :], preferred_element_type=jnp.float32)
          + jnp.dot(sim, wmfc_ref[FH:, :], preferred_element_type=jnp.float32)
          + bmfc_ref[...])
    f = jnp.where(isd, fd, fm)
    f = jnp.where(f > 0, f, jnp.exp(f) - 1.0)
    h = (jnp.dot(f, whfc_ref[...], preferred_element_type=jnp.float32)
         + bhfc_ref[...])
    h_ref[...] = jnp.where(h > 0, h, jnp.exp(h) - 1.0)


def _postb(us, d_sim, m_sim, ts, wdfc, bdfc, wmfc, bmfc, whfc, bhfc):
    full2 = lambda shp: pl.BlockSpec(shp, lambda i: (0, 0))
    ublk = pl.BlockSpec((BR, W), lambda i: (i, 0))
    return pl.pallas_call(
        _postb_body,
        grid=(NBLK,),
        in_specs=[ublk] * 5
        + [pl.BlockSpec((BR, 128), lambda i: (i, 0)),
           pl.BlockSpec((BR, 128), lambda i: (i, 0)),
           full2((8, 128)),
           full2((FH + 128, F)), pl.BlockSpec((1, F), lambda i: (0, 0)),
           full2((FH + 128, F)), pl.BlockSpec((1, F), lambda i: (0, 0)),
           full2((F, F)), pl.BlockSpec((1, F), lambda i: (0, 0))],
        out_specs=pl.BlockSpec((BR, F), lambda i: (i, 0)),
        out_shape=jax.ShapeDtypeStruct((N, F), jnp.float32),
    )(*us, d_sim, m_sim, ts, wdfc, bdfc.reshape(1, F), wmfc,
      bmfc.reshape(1, F), whfc, bhfc.reshape(1, F))


def _final_body(h_ref, dis_ref, mir_ref, wp_ref, out_ref,
                didx, midx, hd, hm, ob, wbuf):
    cid = lax.axis_index("c")
    sid = lax.axis_index("s")
    wid = sid * 2 + cid
    per = NPAIR // 32
    base = wid * per
    pltpu.sync_copy(dis_ref.at[pl.ds(base, per)], didx)
    pltpu.sync_copy(mir_ref.at[pl.ds(base, per)], midx)
    pltpu.sync_copy(wp_ref, wbuf)
    pltpu.sync_copy(h_ref.at[didx], hd)
    pltpu.sync_copy(h_ref.at[midx], hm)
    wv = [wbuf[pl.ds(c * 16, 16)] for c in range(8)]
    bp = wbuf[pl.ds(128, 16)][0]

    def grp(g, _):
        gv = jnp.zeros((16,), jnp.float32)
        for kk in range(16):
            p = g * 16 + kk
            acc = hd[p, pl.ds(0, 16)] * wv[0]
            for c in range(1, 4):
                acc = acc + hd[p, pl.ds(c * 16, 16)] * wv[c]
            for c in range(4):
                acc = acc + hm[p, pl.ds(c * 16, 16)] * wv[4 + c]
            sc = jnp.sum(acc)
            gv = jnp.where(_iota16() == kk, sc, gv)
        ev = jnp.exp(-(gv + bp))
        ob[pl.ds(g * 16, 16)] = 1.0 / (1.0 + ev)
        return 0

    lax.fori_loop(0, per // 16, grp, 0)
    pltpu.sync_copy(ob, out_ref.at[pl.ds(base, per)])


def _final(h, diseases, mirnas, wpb):
    per = NPAIR // 32
    mesh = plsc.VectorSubcoreMesh(core_axis_name="c", subcore_axis_name="s")
    kern = pl.kernel(
        _final_body,
        out_type=jax.ShapeDtypeStruct((NPAIR,), jnp.float32),
        mesh=mesh,
        compiler_params=pltpu.CompilerParams(
            needs_layout_passes=False, use_tc_tiling_on_sc=False),
        scratch_types=[
            pltpu.VMEM((per,), jnp.int32),
            pltpu.VMEM((per,), jnp.int32),
            pltpu.VMEM((per, F), jnp.float32),
            pltpu.VMEM((per, F), jnp.float32),
            pltpu.VMEM((per,), jnp.float32),
            pltpu.VMEM((144,), jnp.float32),
        ],
    )
    return kern(h, diseases, mirnas, wpb)


def kernel(d_sim, m_sim, Wd, Wm, a1, a2, Wmd, Wmm, am1, am2, Ws1, bs1, Ws2,
           Wmfc, bmfc, Wdfc, bdfc, Whfc, bhfc, Wp, bp,
           edge_index0, edge_index_c, edge_index_e, edge_index_t,
           edge_index_g, diseases, mirnas):
    f32 = jnp.float32
    cat = lambda w: w.astype(f32).transpose(1, 0, 2).reshape(128, FH)
    eye = jnp.eye(H, dtype=f32)
    blk = lambda a: (a.astype(f32)[:, :, None] * eye[:, None, :]).reshape(FH, H)
    wda, wma = cat(Wd), cat(Wm)
    wdb, wmb = cat(Wmd), cat(Wmm)
    a1a, a2a = blk(a1), blk(a2)
    a1b, a2b = blk(am1), blk(am2)

    hxa, s2a, hxb, s2b = _prep(d_sim, m_sim, wda, wma, wdb, wmb,
                               a1a, a2a, a1b, a2b)
    pad = lambda s: jnp.pad(s, ((0, NPAD - N), (0, 0))).reshape(-1)
    s2a_f, s2b_f = pad(s2a), pad(s2b)

    edges = [(edge_index0, hxa, s2a_f), (edge_index_c, hxb, s2b_f),
             (edge_index_e, hxb, s2b_f), (edge_index_t, hxb, s2b_f),
             (edge_index_g, hxb, s2b_f)]
    zz = jnp.zeros((RN // 16, W), jnp.float32)
    us = [_gat_edges(hx, s2f, ei[0].astype(jnp.int32), ei[1].astype(jnp.int32),
                     zz)
          for ei, hx, s2f in edges]

    ts = _posta(us, Ws1, bs1, Ws2)
    h = _postb(us, d_sim, m_sim, ts, Wdfc, bdfc, Wmfc, bmfc, Whfc, bhfc)

    wpb = jnp.concatenate([Wp.reshape(-1), bp.reshape(-1),
                           jnp.zeros((15,), f32)])
    out = _final(h, diseases.astype(jnp.int32), mirnas.astype(jnp.int32), wpb)
    return out.reshape(NPAIR, 1)


# speedup vs baseline: 1.0422x; 1.0033x over previous
import functools

import jax
import jax.numpy as jnp
from jax import lax
from jax.experimental import pallas as pl
from jax.experimental.pallas import tpu as pltpu, tpu_sc as plsc

ND = 38300
NM = 49500
N = ND + NM
H = 4
F = 64
FH = F * H
W = 272
RN = 3680
NRANGES = 24
NPAD = NRANGES * RN
RPC = NRANGES // 2
SLOPE = 0.2
BR = 200
NBLK = N // BR
KB = 64
SC_CH = 4096
CAP = 6400
CAPA = CAP + 256
ZB = 12
NPAIR = 16384
EPS = 1e-16


def _iota16():
    return lax.iota(jnp.int32, 16)


def _prep_body(d_ref, m_ref, wda_ref, wma_ref, wdb_ref, wmb_ref,
               a1a_ref, a2a_ref, a1b_ref, a2b_ref,
               hxa_ref, s2a_ref, hxb_ref, s2b_ref):
    i = pl.program_id(0)
    row = i * BR + lax.broadcasted_iota(jnp.int32, (BR, 1), 0)
    isd = row < ND
    d = d_ref[...]
    m = m_ref[...]
    zpad = jnp.zeros((BR, W - FH - H), jnp.float32)
    for wd, wm, a1, a2, hx_ref, s2_ref in (
        (wda_ref, wma_ref, a1a_ref, a2a_ref, hxa_ref, s2a_ref),
        (wdb_ref, wmb_ref, a1b_ref, a2b_ref, hxb_ref, s2b_ref),
    ):
        hd = jnp.dot(d, wd[...], preferred_element_type=jnp.float32)
        hm = jnp.dot(m, wm[...], preferred_element_type=jnp.float32)
        h = jnp.where(isd, hd, hm)
        s1 = jnp.dot(h, a1[...], preferred_element_type=jnp.float32)
        s2 = jnp.dot(h, a2[...], preferred_element_type=jnp.float32)
        hx_ref[...] = jnp.concatenate([h, s1, zpad], axis=1)
        s2_ref[...] = s2


def _prep(d_sim, m_sim, wda, wma, wdb, wmb, a1a, a2a, a1b, a2b):
    full = lambda shp: pl.BlockSpec(shp, lambda i: (0, 0))
    return pl.pallas_call(
        _prep_body,
        grid=(NBLK,),
        in_specs=[
            pl.BlockSpec((BR, 128), lambda i: (i, 0)),
            pl.BlockSpec((BR, 128), lambda i: (i, 0)),
            full((128, FH)), full((128, FH)), full((128, FH)), full((128, FH)),
            full((FH, H)), full((FH, H)), full((FH, H)), full((FH, H)),
        ],
        out_specs=[
            pl.BlockSpec((BR, W), lambda i: (i, 0)),
            pl.BlockSpec((BR, H), lambda i: (i, 0)),
            pl.BlockSpec((BR, W), lambda i: (i, 0)),
            pl.BlockSpec((BR, H), lambda i: (i, 0)),
        ],
        out_shape=[
            jax.ShapeDtypeStruct((N, W), jnp.float32),
            jax.ShapeDtypeStruct((N, H), jnp.float32),
            jax.ShapeDtypeStruct((N, W), jnp.float32),
            jax.ShapeDtypeStruct((N, H), jnp.float32),
        ],
    )(d_sim, m_sim, wda, wma, wdb, wmb, a1a, a2a, a1b, a2b)


def _gat_body_make(E):
    NCH = pl.cdiv(E, SC_CH)
    NJ = pl.cdiv(NCH, 16)
    last_off = E - SC_CH

    def body(hx_ref, s2_ref, src_ref, dst_ref, zz_ref, u_ref,
             sbuf, dbuf, pkc, s2blk, hb0, hb1, idxb0, idxb1, ofsb0, ofsb1,
             sem0, sem1, sem2, sem3, u_acc):
        cid = lax.axis_index("c")
        sid = lax.axis_index("s")
        neg1 = jnp.full((16,), -1, jnp.int32)

        def range_body(ri, _):
            rg = cid * RPC + ri
            base = rg * RN
            row0 = sid * (RN // 16)
            pltpu.sync_copy(zz_ref.at[pl.ds(0, RN // 16)],
                            u_acc.at[pl.ds(row0, RN // 16)])
            pltpu.sync_copy(s2_ref.at[pl.ds(base * H, RN * H)], s2blk)

            def chunk_body(j, k):
                chunk = j * 16 + sid
                cs = chunk * SC_CH
                off = jnp.minimum(cs, last_off)
                pltpu.sync_copy(src_ref.at[pl.ds(off, SC_CH)], sbuf)
                pltpu.sync_copy(dst_ref.at[pl.ds(off, SC_CH)], dbuf)
                lo16 = jnp.minimum(cs - off, SC_CH) // 16

                def pre(t, _):
                    dbuf[pl.ds(t * 16, 16)] = neg1
                    return 0

                lax.fori_loop(0, lo16, pre, 0)

                def group_body(g, k):
                    srcv = sbuf[pl.ds(g * 16, 16)]
                    dstv = dbuf[pl.ds(g * 16, 16)]
                    msk = (dstv >= base) & (dstv < base + RN)
                    inc = jnp.where(msk, jnp.int32(1), jnp.int32(0))
                    kc = jnp.minimum(k, CAP)
                    packed = srcv * 8192 + (dstv - base)
                    _, vs = plsc.sort_key_val(1 - inc, packed)
                    pkc[pl.ds(kc, 16)] = vs
                    return k + jnp.sum(inc)

                return lax.fori_loop(0, SC_CH // 16, group_body, k)

            k = lax.fori_loop(0, NJ, chunk_body, jnp.int32(0))
            k = jnp.minimum(k, CAP)
            plsc.subcore_barrier()

            bufs = ((hb0, idxb0, ofsb0, sem0, sem2),
                    (hb1, idxb1, ofsb1, sem1, sem3))

            def build(b0, idxb, ofsb):
                for g in range(KB // 16):
                    pos = b0 + g * 16 + _iota16()
                    valid = pos < k
                    pv = pkc[pl.ds(b0 + g * 16, 16)]
                    sv = lax.shift_right_logical(pv, 13)
                    ov = jnp.bitwise_and(pv, 8191)
                    idxb[pl.ds(g * 16, 16)] = jnp.where(valid, sv, 0)
                    ofsb[pl.ds(g * 16, 16)] = jnp.where(valid, ov, 0)

            def process(b0, hb, ofsb, ssem):
                for g in range(KB // 16):
                    rowv = g * 16 + _iota16()
                    valid = (b0 + rowv) < k
                    ov = ofsb[pl.ds(g * 16, 16)]
                    for hd in range(H):
                        colv = jnp.full((16,), FH + hd, jnp.int32)
                        s1v = plsc.load_gather(hb, [rowv, colv])
                        s2v = plsc.load_gather(s2blk, [ov * H + hd])
                        e = s1v + s2v
                        e = jnp.where(e >= 0, e, SLOPE * e)
                        eev = jnp.where(valid, jnp.exp(e), 0.0)
                        plsc.store_scatter(hb, [rowv, colv], eev)

                def scale_body(r, _):
                    ev = hb[r, pl.ds(FH, 16)]
                    for hd in range(H):
                        ee = ev[hd]
                        for c in range(hd * 4, hd * 4 + 4):
                            hb[r, pl.ds(c * 16, 16)] = (
                                hb[r, pl.ds(c * 16, 16)] * ee)
                    return 0

                lax.fori_loop(0, KB, scale_body, 0)
                return pltpu.async_copy(hb, u_acc.at[ofsb], ssem, add=True)

            def pair_body(i, _):
                descs = []
                sdescs = []
                nb = len(bufs)
                for par, (hb, idxb, ofsb, sem, ssem) in enumerate(bufs):
                    b0 = (nb * i + par) * KB
                    build(b0, idxb, ofsb)
                    descs.append(pltpu.async_copy(hx_ref.at[idxb], hb, sem))
                for par, (hb, idxb, ofsb, sem, ssem) in enumerate(bufs):
                    b0 = (nb * i + par) * KB
                    descs[par].wait()
                    sdescs.append(process(b0, hb, ofsb, ssem))
                for sd in sdescs:
                    sd.wait()
                return 0

            lax.fori_loop(0, pl.cdiv(k, len(bufs) * KB), pair_body, 0)
            plsc.subcore_barrier()
            for o in range(0, RN // 16, 2 * KB):
                sz = min(2 * KB, RN // 16 - o)
                pltpu.sync_copy(u_acc.at[pl.ds(row0 + o, sz)],
                                u_ref.at[pl.ds(base + row0 + o, sz)])
            return 0

        lax.fori_loop(0, RPC, range_body, 0)

    return body


def _gat_edges(hx, s2flat, src, dst, zz):
    E = src.shape[0]
    mesh = plsc.VectorSubcoreMesh(core_axis_name="c", subcore_axis_name="s")
    kern = pl.kernel(
        _gat_body_make(E),
        out_type=jax.ShapeDtypeStruct((NPAD, W), jnp.float32),
        mesh=mesh,
        compiler_params=pltpu.CompilerParams(
            needs_layout_passes=False, use_tc_tiling_on_sc=False),
        scratch_types=[
            pltpu.VMEM((SC_CH,), jnp.int32),
            pltpu.VMEM((SC_CH,), jnp.int32),
            pltpu.VMEM((CAPA,), jnp.int32),
            pltpu.VMEM((RN * H,), jnp.float32),
            pltpu.VMEM((KB, W), jnp.float32),
            pltpu.VMEM((KB, W), jnp.float32),
            pltpu.VMEM((KB,), jnp.int32),
            pltpu.VMEM((KB,), jnp.int32),
            pltpu.VMEM((KB,), jnp.int32),
            pltpu.VMEM((KB,), jnp.int32),
            pltpu.SemaphoreType.DMA,
            pltpu.SemaphoreType.DMA,
            pltpu.SemaphoreType.DMA,
            pltpu.SemaphoreType.DMA,
            pltpu.VMEM_SHARED((RN, W), jnp.float32),
        ],
    )
    return kern(hx, s2flat, src, dst, zz)


def _posta_body(u0, u1, u2, u3, u4, ws1_ref, bs1_ref, ws2_ref, ts_ref):
    i = pl.program_id(0)

    @pl.when(i == 0)
    def _():
        ts_ref[...] = jnp.zeros((8, 128), jnp.float32)

    row = i * BR + lax.broadcasted_iota(jnp.int32, (BR, 1), 0)
    isd = row < ND
    ws1 = ws1_ref[...]
    bs1 = bs1_ref[...]
    ws2 = ws2_ref[...]
    lane = lax.broadcasted_iota(jnp.int32, (8, 128), 1)
    prow = lax.broadcasted_iota(jnp.int32, (8, 128), 0)
    acc = jnp.zeros((8, 128), jnp.float32)
    for p, u_ref in enumerate((u0, u1, u2, u3, u4)):
        y = _u_to_y(u_ref[...])
        t = jnp.dot(jnp.tanh(jnp.dot(y, ws1,
                                     preferred_element_type=jnp.float32)
                             + bs1),
                    ws2, preferred_element_type=jnp.float32)
        td = jnp.sum(jnp.where(isd, t, 0.0))
        tm = jnp.sum(jnp.where(isd, 0.0, t))
        val = jnp.where(lane == 0, td, jnp.where(lane == 1, tm, 0.0))
        acc = acc + jnp.where(prow == p, val, 0.0)
    ts_ref[...] = ts_ref[...] + acc


def _u_to_y(u):
    s = u[:, FH:FH + H]
    den = jnp.concatenate(
        [jnp.broadcast_to(s[:, hd:hd + 1], (BR, F)) for hd in range(H)],
        axis=1) + EPS
    y = u[:, :FH] / den
    return jnp.where(y > 0, y, jnp.exp(y) - 1.0)


def _posta(us, ws1, bs1, ws2):
    full2 = lambda shp: pl.BlockSpec(shp, lambda i: (0, 0))
    ublk = pl.BlockSpec((BR, W), lambda i: (i, 0))
    return pl.pallas_call(
        _posta_body,
        grid=(NBLK,),
        in_specs=[ublk] * 5 + [full2((FH, 128)),
                               pl.BlockSpec((1, 128), lambda i: (0, 0)),
                               full2((128, 1))],
        out_specs=full2((8, 128)),
        out_shape=jax.ShapeDtypeStruct((8, 128), jnp.float32),
    )(*us, ws1, bs1.reshape(1, 128), ws2)


def _postb_body(u0, u1, u2, u3, u4, d_ref, m_ref, ts_ref,
                wdfc_ref, bdfc_ref, wmfc_ref, bmfc_ref, whfc_ref, bhfc_ref,
                h_ref):
    i = pl.program_id(0)
    row = i * BR + lax.broadcasted_iota(jnp.int32, (BR, 1), 0)
    isd = row < ND
    ts = ts_ref[...]
    h1 = jnp.zeros((BR, FH), jnp.float32)
    for p, u_ref in enumerate((u0, u1, u2, u3, u4)):
        betad = jax.nn.sigmoid(ts[p, 0] / ND)
        betam = jax.nn.sigmoid(ts[p, 1] / NM)
        beta = jnp.where(isd, betad, betam)
        h1 = h1 + beta * _u_to_y(u_ref[...])
    sim = jnp.where(isd, d_ref[...], m_ref[...])
    fd = (jnp.dot(h1, wdfc_ref[:FH, :], preferred_element_type=jnp.float32)
          + jnp.dot(sim, wdfc_ref[FH:, :], preferred_element_type=jnp.float32)
          + bdfc_ref[...])
    fm = (jnp.dot(h1, wmfc_ref[:FH, :], preferred_element_type=jnp.float32)
          + jnp.dot(sim, wmfc_ref[FH:, :], preferred_element_type=jnp.float32)
          + bmfc_ref[...])
    f = jnp.where(isd, fd, fm)
    f = jnp.where(f > 0, f, jnp.exp(f) - 1.0)
    h = (jnp.dot(f, whfc_ref[...], preferred_element_type=jnp.float32)
         + bhfc_ref[...])
    h_ref[...] = jnp.where(h > 0, h, jnp.exp(h) - 1.0)


def _postb(us, d_sim, m_sim, ts, wdfc, bdfc, wmfc, bmfc, whfc, bhfc):
    full2 = lambda shp: pl.BlockSpec(shp, lambda i: (0, 0))
    ublk = pl.BlockSpec((BR, W), lambda i: (i, 0))
    return pl.pallas_call(
        _postb_body,
        grid=(NBLK,),
        in_specs=[ublk] * 5
        + [pl.BlockSpec((BR, 128), lambda i: (i, 0)),
           pl.BlockSpec((BR, 128), lambda i: (i, 0)),
           full2((8, 128)),
           full2((FH + 128, F)), pl.BlockSpec((1, F), lambda i: (0, 0)),
           full2((FH + 128, F)), pl.BlockSpec((1, F), lambda i: (0, 0)),
           full2((F, F)), pl.BlockSpec((1, F), lambda i: (0, 0))],
        out_specs=pl.BlockSpec((BR, F), lambda i: (i, 0)),
        out_shape=jax.ShapeDtypeStruct((N, F), jnp.float32),
    )(*us, d_sim, m_sim, ts, wdfc, bdfc.reshape(1, F), wmfc,
      bmfc.reshape(1, F), whfc, bhfc.reshape(1, F))


def _final_body(h_ref, dis_ref, mir_ref, wp_ref, out_ref,
                didx, midx, hd, hm, ob, wbuf):
    cid = lax.axis_index("c")
    sid = lax.axis_index("s")
    wid = sid * 2 + cid
    per = NPAIR // 32
    base = wid * per
    pltpu.sync_copy(dis_ref.at[pl.ds(base, per)], didx)
    pltpu.sync_copy(mir_ref.at[pl.ds(base, per)], midx)
    pltpu.sync_copy(wp_ref, wbuf)
    pltpu.sync_copy(h_ref.at[didx], hd)
    pltpu.sync_copy(h_ref.at[midx], hm)
    wv = [wbuf[pl.ds(c * 16, 16)] for c in range(8)]
    bp = wbuf[pl.ds(128, 16)][0]

    def grp(g, _):
        gv = jnp.zeros((16,), jnp.float32)
        for kk in range(16):
            p = g * 16 + kk
            acc = hd[p, pl.ds(0, 16)] * wv[0]
            for c in range(1, 4):
                acc = acc + hd[p, pl.ds(c * 16, 16)] * wv[c]
            for c in range(4):
                acc = acc + hm[p, pl.ds(c * 16, 16)] * wv[4 + c]
            sc = jnp.sum(acc)
            gv = jnp.where(_iota16() == kk, sc, gv)
        ev = jnp.exp(-(gv + bp))
        ob[pl.ds(g * 16, 16)] = 1.0 / (1.0 + ev)
        return 0

    lax.fori_loop(0, per // 16, grp, 0)
    pltpu.sync_copy(ob, out_ref.at[pl.ds(base, per)])


def _final(h, diseases, mirnas, wpb):
    per = NPAIR // 32
    mesh = plsc.VectorSubcoreMesh(core_axis_name="c", subcore_axis_name="s")
    kern = pl.kernel(
        _final_body,
        out_type=jax.ShapeDtypeStruct((NPAIR,), jnp.float32),
        mesh=mesh,
        compiler_params=pltpu.CompilerParams(
            needs_layout_passes=False, use_tc_tiling_on_sc=False),
        scratch_types=[
            pltpu.VMEM((per,), jnp.int32),
            pltpu.VMEM((per,), jnp.int32),
            pltpu.VMEM((per, F), jnp.float32),
            pltpu.VMEM((per, F), jnp.float32),
            pltpu.VMEM((per,), jnp.float32),
            pltpu.VMEM((144,), jnp.float32),
        ],
    )
    return kern(h, diseases, mirnas, wpb)


def kernel(d_sim, m_sim, Wd, Wm, a1, a2, Wmd, Wmm, am1, am2, Ws1, bs1, Ws2,
           Wmfc, bmfc, Wdfc, bdfc, Whfc, bhfc, Wp, bp,
           edge_index0, edge_index_c, edge_index_e, edge_index_t,
           edge_index_g, diseases, mirnas):
    f32 = jnp.float32
    cat = lambda w: w.astype(f32).transpose(1, 0, 2).reshape(128, FH)
    eye = jnp.eye(H, dtype=f32)
    blk = lambda a: (a.astype(f32)[:, :, None] * eye[:, None, :]).reshape(FH, H)
    wda, wma = cat(Wd), cat(Wm)
    wdb, wmb = cat(Wmd), cat(Wmm)
    a1a, a2a = blk(a1), blk(a2)
    a1b, a2b = blk(am1), blk(am2)

    hxa, s2a, hxb, s2b = _prep(d_sim, m_sim, wda, wma, wdb, wmb,
                               a1a, a2a, a1b, a2b)
    pad = lambda s: jnp.pad(s, ((0, NPAD - N), (0, 0))).reshape(-1)
    s2a_f, s2b_f = pad(s2a), pad(s2b)

    edges = [(edge_index0, hxa, s2a_f), (edge_index_c, hxb, s2b_f),
             (edge_index_e, hxb, s2b_f), (edge_index_t, hxb, s2b_f),
             (edge_index_g, hxb, s2b_f)]
    zz = jnp.zeros((RN // 16, W), jnp.float32)
    us = [_gat_edges(hx, s2f, ei[0].astype(jnp.int32), ei[1].astype(jnp.int32),
                     zz)
          for ei, hx, s2f in edges]

    ts = _posta(us, Ws1, bs1, Ws2)
    h = _postb(us, d_sim, m_sim, ts, Wdfc, bdfc, Wmfc, bmfc, Whfc, bhfc)

    wpb = jnp.concatenate([Wp.reshape(-1), bp.reshape(-1),
                           jnp.zeros((15,), f32)])
    out = _final(h, diseases.astype(jnp.int32), mirnas.astype(jnp.int32), wpb)
    return out.reshape(NPAIR, 1)


# speedup vs baseline: 1.0869x; 1.0429x over previous
import functools

import jax
import jax.numpy as jnp
from jax import lax
from jax.experimental import pallas as pl
from jax.experimental.pallas import tpu as pltpu, tpu_sc as plsc

ND = 38300
NM = 49500
N = ND + NM
H = 4
F = 64
FH = F * H
W = 272
RN = 3680
NRANGES = 24
NPAD = NRANGES * RN
RPC = NRANGES // 2
SLOPE = 0.2
BR = 200
NBLK = N // BR
KB = 64
SC_CH = 2048
CAP = 6400
CAPA = CAP + 256
ZB = 12
NPAIR = 16384
EPS = 1e-16


def _iota16():
    return lax.iota(jnp.int32, 16)


def _prep_body(d_ref, m_ref, wda_ref, wma_ref, wdb_ref, wmb_ref,
               a1a_ref, a2a_ref, a1b_ref, a2b_ref,
               hxa_ref, s2a_ref, hxb_ref, s2b_ref):
    i = pl.program_id(0)
    row = i * BR + lax.broadcasted_iota(jnp.int32, (BR, 1), 0)
    isd = row < ND
    d = d_ref[...]
    m = m_ref[...]
    zpad = jnp.zeros((BR, W - FH - H), jnp.float32)
    for wd, wm, a1, a2, hx_ref, s2_ref in (
        (wda_ref, wma_ref, a1a_ref, a2a_ref, hxa_ref, s2a_ref),
        (wdb_ref, wmb_ref, a1b_ref, a2b_ref, hxb_ref, s2b_ref),
    ):
        hd = jnp.dot(d, wd[...], preferred_element_type=jnp.float32)
        hm = jnp.dot(m, wm[...], preferred_element_type=jnp.float32)
        h = jnp.where(isd, hd, hm)
        s1 = jnp.dot(h, a1[...], preferred_element_type=jnp.float32)
        s2 = jnp.dot(h, a2[...], preferred_element_type=jnp.float32)
        hx_ref[...] = jnp.concatenate([h, s1, zpad], axis=1)
        s2_ref[...] = s2


def _prep(d_sim, m_sim, wda, wma, wdb, wmb, a1a, a2a, a1b, a2b):
    full = lambda shp: pl.BlockSpec(shp, lambda i: (0, 0))
    return pl.pallas_call(
        _prep_body,
        grid=(NBLK,),
        in_specs=[
            pl.BlockSpec((BR, 128), lambda i: (i, 0)),
            pl.BlockSpec((BR, 128), lambda i: (i, 0)),
            full((128, FH)), full((128, FH)), full((128, FH)), full((128, FH)),
            full((FH, H)), full((FH, H)), full((FH, H)), full((FH, H)),
        ],
        out_specs=[
            pl.BlockSpec((BR, W), lambda i: (i, 0)),
            pl.BlockSpec((BR, H), lambda i: (i, 0)),
            pl.BlockSpec((BR, W), lambda i: (i, 0)),
            pl.BlockSpec((BR, H), lambda i: (i, 0)),
        ],
        out_shape=[
            jax.ShapeDtypeStruct((N, W), jnp.float32),
            jax.ShapeDtypeStruct((N, H), jnp.float32),
            jax.ShapeDtypeStruct((N, W), jnp.float32),
            jax.ShapeDtypeStruct((N, H), jnp.float32),
        ],
    )(d_sim, m_sim, wda, wma, wdb, wmb, a1a, a2a, a1b, a2b)


def _gat_body_make(E):
    NCH = pl.cdiv(E, SC_CH)
    NJ = pl.cdiv(NCH, 16)
    last_off = E - SC_CH

    def body(hx_ref, s2_ref, src_ref, dst_ref, zz_ref, u_ref,
             sb0, db0, sb1, db1, pkc, s2blk, hb0, hb1, idxb0, idxb1,
             ofsb0, ofsb1, sem0, sem1, sem2, sem3, sem4, sem5, sem6, sem7,
             u_acc):
        cid = lax.axis_index("c")
        sid = lax.axis_index("s")
        neg1 = jnp.full((16,), -1, jnp.int32)

        def range_body(ri, _):
            rg = cid * RPC + ri
            base = rg * RN
            row0 = sid * (RN // 16)
            pltpu.sync_copy(zz_ref.at[pl.ds(0, RN // 16)],
                            u_acc.at[pl.ds(row0, RN // 16)])
            pltpu.sync_copy(s2_ref.at[pl.ds(base * H, RN * H)], s2blk)

            cbufs = ((sb0, db0, sem4, sem5), (sb1, db1, sem6, sem7))

            def chunk_pair(jp, k):
                descs = []
                for par, (sb, db, sse, dse) in enumerate(cbufs):
                    chunk = (2 * jp + par) * 16 + sid
                    cs = chunk * SC_CH
                    off = jnp.minimum(cs, last_off)
                    descs.append(
                        (pltpu.async_copy(src_ref.at[pl.ds(off, SC_CH)],
                                          sb, sse),
                         pltpu.async_copy(dst_ref.at[pl.ds(off, SC_CH)],
                                          db, dse)))
                for par, (sb, db, sse, dse) in enumerate(cbufs):
                    chunk = (2 * jp + par) * 16 + sid
                    cs = chunk * SC_CH
                    off = jnp.minimum(cs, last_off)
                    descs[par][0].wait()
                    descs[par][1].wait()
                    lo16 = jnp.minimum(cs - off, SC_CH) // 16

                    def pre(t, _, db=db):
                        db[pl.ds(t * 16, 16)] = neg1
                        return 0

                    lax.fori_loop(0, lo16, pre, 0)

                    def group_body(g, k, sb=sb, db=db):
                        srcv = sb[pl.ds(g * 16, 16)]
                        dstv = db[pl.ds(g * 16, 16)]
                        msk = (dstv >= base) & (dstv < base + RN)
                        inc = jnp.where(msk, jnp.int32(1), jnp.int32(0))
                        kc = jnp.minimum(k, CAP)
                        packed = srcv * 8192 + (dstv - base)
                        _, vs = plsc.sort_key_val(1 - inc, packed)
                        pkc[pl.ds(kc, 16)] = vs
                        return k + jnp.sum(inc)

                    k = lax.fori_loop(0, SC_CH // 16, group_body, k)
                return k

            k = lax.fori_loop(0, pl.cdiv(NJ, 2), chunk_pair, jnp.int32(0))
            k = jnp.minimum(k, CAP)
            plsc.subcore_barrier()

            bufs = ((hb0, idxb0, ofsb0, sem0, sem2),
                    (hb1, idxb1, ofsb1, sem1, sem3))

            def build(b0, idxb, ofsb):
                for g in range(KB // 16):
                    pos = b0 + g * 16 + _iota16()
                    valid = pos < k
                    pv = pkc[pl.ds(b0 + g * 16, 16)]
                    sv = lax.shift_right_logical(pv, 13)
                    ov = jnp.bitwise_and(pv, 8191)
                    idxb[pl.ds(g * 16, 16)] = jnp.where(valid, sv, 0)
                    ofsb[pl.ds(g * 16, 16)] = jnp.where(valid, ov, 0)

            def process(b0, hb, ofsb, ssem):
                for g in range(KB // 16):
                    rowv = g * 16 + _iota16()
                    valid = (b0 + rowv) < k
                    ov = ofsb[pl.ds(g * 16, 16)]
                    for hd in range(H):
                        colv = jnp.full((16,), FH + hd, jnp.int32)
                        s1v = plsc.load_gather(hb, [rowv, colv])
                        s2v = plsc.load_gather(s2blk, [ov * H + hd])
                        e = s1v + s2v
                        e = jnp.where(e >= 0, e, SLOPE * e)
                        eev = jnp.where(valid, jnp.exp(e), 0.0)
                        plsc.store_scatter(hb, [rowv, colv], eev)

                def scale_body(r, _):
                    ev = hb[r, pl.ds(FH, 16)]
                    for hd in range(H):
                        ee = ev[hd]
                        for c in range(hd * 4, hd * 4 + 4):
                            hb[r, pl.ds(c * 16, 16)] = (
                                hb[r, pl.ds(c * 16, 16)] * ee)
                    return 0

                lax.fori_loop(0, KB, scale_body, 0)
                return pltpu.async_copy(hb, u_acc.at[ofsb], ssem, add=True)

            def pair_body(i, _):
                descs = []
                sdescs = []
                nb = len(bufs)
                for par, (hb, idxb, ofsb, sem, ssem) in enumerate(bufs):
                    b0 = (nb * i + par) * KB
                    build(b0, idxb, ofsb)
                    descs.append(pltpu.async_copy(hx_ref.at[idxb], hb, sem))
                for par, (hb, idxb, ofsb, sem, ssem) in enumerate(bufs):
                    b0 = (nb * i + par) * KB
                    descs[par].wait()
                    sdescs.append(process(b0, hb, ofsb, ssem))
                for sd in sdescs:
                    sd.wait()
                return 0

            lax.fori_loop(0, pl.cdiv(k, len(bufs) * KB), pair_body, 0)
            plsc.subcore_barrier()
            for o in range(0, RN // 16, 2 * KB):
                sz = min(2 * KB, RN // 16 - o)
                pltpu.sync_copy(u_acc.at[pl.ds(row0 + o, sz)],
                                u_ref.at[pl.ds(base + row0 + o, sz)])
            return 0

        lax.fori_loop(0, RPC, range_body, 0)

    return body


def _gat_edges(hx, s2flat, src, dst, zz):
    E = src.shape[0]
    mesh = plsc.VectorSubcoreMesh(core_axis_name="c", subcore_axis_name="s")
    kern = pl.kernel(
        _gat_body_make(E),
        out_type=jax.ShapeDtypeStruct((NPAD, W), jnp.float32),
        mesh=mesh,
        compiler_params=pltpu.CompilerParams(
            needs_layout_passes=False, use_tc_tiling_on_sc=False),
        scratch_types=[
            pltpu.VMEM((SC_CH,), jnp.int32),
            pltpu.VMEM((SC_CH,), jnp.int32),
            pltpu.VMEM((SC_CH,), jnp.int32),
            pltpu.VMEM((SC_CH,), jnp.int32),
            pltpu.VMEM((CAPA,), jnp.int32),
            pltpu.VMEM((RN * H,), jnp.float32),
            pltpu.VMEM((KB, W), jnp.float32),
            pltpu.VMEM((KB, W), jnp.float32),
            pltpu.VMEM((KB,), jnp.int32),
            pltpu.VMEM((KB,), jnp.int32),
            pltpu.VMEM((KB,), jnp.int32),
            pltpu.VMEM((KB,), jnp.int32),
            pltpu.SemaphoreType.DMA,
            pltpu.SemaphoreType.DMA,
            pltpu.SemaphoreType.DMA,
            pltpu.SemaphoreType.DMA,
            pltpu.SemaphoreType.DMA,
            pltpu.SemaphoreType.DMA,
            pltpu.SemaphoreType.DMA,
            pltpu.SemaphoreType.DMA,
            pltpu.VMEM_SHARED((RN, W), jnp.float32),
        ],
    )
    return kern(hx, s2flat, src, dst, zz)


def _posta_body(u0, u1, u2, u3, u4, ws1_ref, bs1_ref, ws2_ref, ts_ref):
    i = pl.program_id(0)

    @pl.when(i == 0)
    def _():
        ts_ref[...] = jnp.zeros((8, 128), jnp.float32)

    row = i * BR + lax.broadcasted_iota(jnp.int32, (BR, 1), 0)
    isd = row < ND
    ws1 = ws1_ref[...]
    bs1 = bs1_ref[...]
    ws2 = ws2_ref[...]
    lane = lax.broadcasted_iota(jnp.int32, (8, 128), 1)
    prow = lax.broadcasted_iota(jnp.int32, (8, 128), 0)
    acc = jnp.zeros((8, 128), jnp.float32)
    for p, u_ref in enumerate((u0, u1, u2, u3, u4)):
        y = _u_to_y(u_ref[...])
        t = jnp.dot(jnp.tanh(jnp.dot(y, ws1,
                                     preferred_element_type=jnp.float32)
                             + bs1),
                    ws2, preferred_element_type=jnp.float32)
        td = jnp.sum(jnp.where(isd, t, 0.0))
        tm = jnp.sum(jnp.where(isd, 0.0, t))
        val = jnp.where(lane == 0, td, jnp.where(lane == 1, tm, 0.0))
        acc = acc + jnp.where(prow == p, val, 0.0)
    ts_ref[...] = ts_ref[...] + acc


def _u_to_y(u):
    s = u[:, FH:FH + H]
    den = jnp.concatenate(
        [jnp.broadcast_to(s[:, hd:hd + 1], (BR, F)) for hd in range(H)],
        axis=1) + EPS
    y = u[:, :FH] / den
    return jnp.where(y > 0, y, jnp.exp(y) - 1.0)


def _posta(us, ws1, bs1, ws2):
    full2 = lambda shp: pl.BlockSpec(shp, lambda i: (0, 0))
    ublk = pl.BlockSpec((BR, W), lambda i: (i, 0))
    return pl.pallas_call(
        _posta_body,
        grid=(NBLK,),
        in_specs=[ublk] * 5 + [full2((FH, 128)),
                               pl.BlockSpec((1, 128), lambda i: (0, 0)),
                               full2((128, 1))],
        out_specs=full2((8, 128)),
        out_shape=jax.ShapeDtypeStruct((8, 128), jnp.float32),
    )(*us, ws1, bs1.reshape(1, 128), ws2)


def _postb_body(u0, u1, u2, u3, u4, d_ref, m_ref, ts_ref,
                wdfc_ref, bdfc_ref, wmfc_ref, bmfc_ref, whfc_ref, bhfc_ref,
                h_ref):
    i = pl.program_id(0)
    row = i * BR + lax.broadcasted_iota(jnp.int32, (BR, 1), 0)
    isd = row < ND
    ts = ts_ref[...]
    h1 = jnp.zeros((BR, FH), jnp.float32)
    for p, u_ref in enumerate((u0, u1, u2, u3, u4)):
        betad = jax.nn.sigmoid(ts[p, 0] / ND)
        betam = jax.nn.sigmoid(ts[p, 1] / NM)
        beta = jnp.where(isd, betad, betam)
        h1 = h1 + beta * _u_to_y(u_ref[...])
    sim = jnp.where(isd, d_ref[...], m_ref[...])
    fd = (jnp.dot(h1, wdfc_ref[:FH, :], preferred_element_type=jnp.float32)
          + jnp.dot(sim, wdfc_ref[FH:, :], preferred_element_type=jnp.float32)
          + bdfc_ref[...])
    fm = (jnp.dot(h1, wmfc_ref[:FH, :], preferred_element_type=jnp.float32)
          + jnp.dot(sim, wmfc_ref[FH:, :], preferred_element_type=jnp.float32)
          + bmfc_ref[...])
    f = jnp.where(isd, fd, fm)
    f = jnp.where(f > 0, f, jnp.exp(f) - 1.0)
    h = (jnp.dot(f, whfc_ref[...], preferred_element_type=jnp.float32)
         + bhfc_ref[...])
    h_ref[...] = jnp.where(h > 0, h, jnp.exp(h) - 1.0)


def _postb(us, d_sim, m_sim, ts, wdfc, bdfc, wmfc, bmfc, whfc, bhfc):
    full2 = lambda shp: pl.BlockSpec(shp, lambda i: (0, 0))
    ublk = pl.BlockSpec((BR, W), lambda i: (i, 0))
    return pl.pallas_call(
        _postb_body,
        grid=(NBLK,),
        in_specs=[ublk] * 5
        + [pl.BlockSpec((BR, 128), lambda i: (i, 0)),
           pl.BlockSpec((BR, 128), lambda i: (i, 0)),
           full2((8, 128)),
           full2((FH + 128, F)), pl.BlockSpec((1, F), lambda i: (0, 0)),
           full2((FH + 128, F)), pl.BlockSpec((1, F), lambda i: (0, 0)),
           full2((F, F)), pl.BlockSpec((1, F), lambda i: (0, 0))],
        out_specs=pl.BlockSpec((BR, F), lambda i: (i, 0)),
        out_shape=jax.ShapeDtypeStruct((N, F), jnp.float32),
    )(*us, d_sim, m_sim, ts, wdfc, bdfc.reshape(1, F), wmfc,
      bmfc.reshape(1, F), whfc, bhfc.reshape(1, F))


def _final_body(h_ref, dis_ref, mir_ref, wp_ref, out_ref,
                didx, midx, hd, hm, ob, wbuf):
    cid = lax.axis_index("c")
    sid = lax.axis_index("s")
    wid = sid * 2 + cid
    per = NPAIR // 32
    base = wid * per
    pltpu.sync_copy(dis_ref.at[pl.ds(base, per)], didx)
    pltpu.sync_copy(mir_ref.at[pl.ds(base, per)], midx)
    pltpu.sync_copy(wp_ref, wbuf)
    pltpu.sync_copy(h_ref.at[didx], hd)
    pltpu.sync_copy(h_ref.at[midx], hm)
    wv = [wbuf[pl.ds(c * 16, 16)] for c in range(8)]
    bp = wbuf[pl.ds(128, 16)][0]

    def grp(g, _):
        gv = jnp.zeros((16,), jnp.float32)
        for kk in range(16):
            p = g * 16 + kk
            acc = hd[p, pl.ds(0, 16)] * wv[0]
            for c in range(1, 4):
                acc = acc + hd[p, pl.ds(c * 16, 16)] * wv[c]
            for c in range(4):
                acc = acc + hm[p, pl.ds(c * 16, 16)] * wv[4 + c]
            sc = jnp.sum(acc)
            gv = jnp.where(_iota16() == kk, sc, gv)
        ev = jnp.exp(-(gv + bp))
        ob[pl.ds(g * 16, 16)] = 1.0 / (1.0 + ev)
        return 0

    lax.fori_loop(0, per // 16, grp, 0)
    pltpu.sync_copy(ob, out_ref.at[pl.ds(base, per)])


def _final(h, diseases, mirnas, wpb):
    per = NPAIR // 32
    mesh = plsc.VectorSubcoreMesh(core_axis_name="c", subcore_axis_name="s")
    kern = pl.kernel(
        _final_body,
        out_type=jax.ShapeDtypeStruct((NPAIR,), jnp.float32),
        mesh=mesh,
        compiler_params=pltpu.CompilerParams(
            needs_layout_passes=False, use_tc_tiling_on_sc=False),
        scratch_types=[
            pltpu.VMEM((per,), jnp.int32),
            pltpu.VMEM((per,), jnp.int32),
            pltpu.VMEM((per, F), jnp.float32),
            pltpu.VMEM((per, F), jnp.float32),
            pltpu.VMEM((per,), jnp.float32),
            pltpu.VMEM((144,), jnp.float32),
        ],
    )
    return kern(h, diseases, mirnas, wpb)


def kernel(d_sim, m_sim, Wd, Wm, a1, a2, Wmd, Wmm, am1, am2, Ws1, bs1, Ws2,
           Wmfc, bmfc, Wdfc, bdfc, Whfc, bhfc, Wp, bp,
           edge_index0, edge_index_c, edge_index_e, edge_index_t,
           edge_index_g, diseases, mirnas):
    f32 = jnp.float32
    cat = lambda w: w.astype(f32).transpose(1, 0, 2).reshape(128, FH)
    eye = jnp.eye(H, dtype=f32)
    blk = lambda a: (a.astype(f32)[:, :, None] * eye[:, None, :]).reshape(FH, H)
    wda, wma = cat(Wd), cat(Wm)
    wdb, wmb = cat(Wmd), cat(Wmm)
    a1a, a2a = blk(a1), blk(a2)
    a1b, a2b = blk(am1), blk(am2)

    hxa, s2a, hxb, s2b = _prep(d_sim, m_sim, wda, wma, wdb, wmb,
                               a1a, a2a, a1b, a2b)
    pad = lambda s: jnp.pad(s, ((0, NPAD - N), (0, 0))).reshape(-1)
    s2a_f, s2b_f = pad(s2a), pad(s2b)

    edges = [(edge_index0, hxa, s2a_f), (edge_index_c, hxb, s2b_f),
             (edge_index_e, hxb, s2b_f), (edge_index_t, hxb, s2b_f),
             (edge_index_g, hxb, s2b_f)]
    zz = jnp.zeros((RN // 16, W), jnp.float32)
    us = [_gat_edges(hx, s2f, ei[0].astype(jnp.int32), ei[1].astype(jnp.int32),
                     zz)
          for ei, hx, s2f in edges]

    ts = _posta(us, Ws1, bs1, Ws2)
    h = _postb(us, d_sim, m_sim, ts, Wdfc, bdfc, Wmfc, bmfc, Whfc, bhfc)

    wpb = jnp.concatenate([Wp.reshape(-1), bp.reshape(-1),
                           jnp.zeros((15,), f32)])
    out = _final(h, diseases.astype(jnp.int32), mirnas.astype(jnp.int32), wpb)
    return out.reshape(NPAIR, 1)
